# baseline (speedup 1.0000x reference)
import functools

import jax
import jax.numpy as jnp
from jax.experimental import pallas as pl
from jax.experimental.pallas import tpu as pltpu

IN_DIM = 6
N_PTS = 4096
K_NBR = 64

_BIG_I = 1 << 30


def _fps_kernel(px_ref, py_ref, pz_ref, ox_ref, oy_ref, oz_ref, *, n_samples):
    px = px_ref[:, :]
    py = py_ref[:, :]
    pz = pz_ref[:, :]
    shape = px.shape
    iota = jax.lax.broadcasted_iota(jnp.int32, shape, 0) * 128 + jax.lax.broadcasted_iota(jnp.int32, shape, 1)

    x0 = px[0, 0]
    y0 = py[0, 0]
    z0 = pz[0, 0]
    ox_ref[0:1, :] = x0.reshape(1, 1)
    oy_ref[0:1, :] = y0.reshape(1, 1)
    oz_ref[0:1, :] = z0.reshape(1, 1)
    dx = px - x0
    dy = py - y0
    dz = pz - z0
    d = (dx * dx + dy * dy) + dz * dz

    def body(i, d):
        m = jnp.max(d)
        nxt = jnp.min(jnp.where(d == m, iota, _BIG_I))
        msk = iota == nxt
        xn = jnp.sum(jnp.where(msk, px, 0.0))
        yn = jnp.sum(jnp.where(msk, py, 0.0))
        zn = jnp.sum(jnp.where(msk, pz, 0.0))
        ox_ref[pl.ds(i, 1), :] = xn.reshape(1, 1)
        oy_ref[pl.ds(i, 1), :] = yn.reshape(1, 1)
        oz_ref[pl.ds(i, 1), :] = zn.reshape(1, 1)
        ex = px - xn
        ey = py - yn
        ez = pz - zn
        nd = (ex * ex + ey * ey) + ez * ez
        return jnp.minimum(d, nd)

    jax.lax.fori_loop(1, n_samples, body, d)


def _fps_pos(pos, n_samples, interpret=False):
    n = pos.shape[0]
    r = n // 128
    px = pos[:, 0].reshape(r, 128)
    py = pos[:, 1].reshape(r, 128)
    pz = pos[:, 2].reshape(r, 128)
    out = pl.pallas_call(
        functools.partial(_fps_kernel, n_samples=n_samples),
        out_shape=[jax.ShapeDtypeStruct((n_samples, 1), jnp.float32)] * 3,
        interpret=interpret,
    )(px, py, pz)
    return jnp.concatenate(out, axis=1)


def _mlp(h, params):
    for W, b in params:
        h = jax.nn.relu(h @ W + b)
    return h


def _dot(a, b):
    return jnp.dot(a, b, preferred_element_type=jnp.float32,
                   precision=jax.lax.Precision.HIGHEST)


def _fp_knn3_kernel(*refs, n_layers):
    (psx_ref, psy_ref, psz_ref, xsrc_ref, ptx_ref, pty_ref, ptz_ref,
     xskip_ref, w0a_ref, w0b_ref, b0_ref) = refs[:11]
    layer_refs = refs[11:11 + 2 * (n_layers - 1)]
    out_ref = refs[11 + 2 * (n_layers - 1)]

    dx = ptx_ref[:, :] - psx_ref[:, :]
    dy = pty_ref[:, :] - psy_ref[:, :]
    dz = ptz_ref[:, :] - psz_ref[:, :]
    d2 = (dx * dx + dy * dy) + dz * dz
    m1 = jnp.min(d2, axis=1, keepdims=True)
    t = jnp.where(d2 == m1, jnp.inf, d2)
    m2 = jnp.min(t, axis=1, keepdims=True)
    t = jnp.where(t == m2, jnp.inf, t)
    m3 = jnp.min(t, axis=1, keepdims=True)
    w = jnp.where(d2 <= m3, 1.0 / jnp.maximum(d2, 1e-16), 0.0)
    sw = jnp.sum(w, axis=1, keepdims=True)
    y = _dot(w, xsrc_ref[:, :]) / sw
    h = jnp.maximum(_dot(y, w0a_ref[:, :]) + _dot(xskip_ref[:, :], w0b_ref[:, :]) + b0_ref[:, :], 0.0)
    for li in range(n_layers - 1):
        w_ref, b_ref = layer_refs[2 * li], layer_refs[2 * li + 1]
        h = jnp.maximum(_dot(h, w_ref[:, :]) + b_ref[:, :], 0.0)
    out_ref[:, :] = h


def _fp_knn3(pos_src, x_src, pos_t, x_skip, params, block_t, interpret=False):
    s = pos_src.shape[0]
    t_n = pos_t.shape[0]
    c = x_src.shape[1]
    cs = x_skip.shape[1]
    w0, b0 = params[0]
    w0a, w0b = w0[:c], w0[c:]
    rest = params[1:]
    n_layers = len(params)
    cout = params[-1][0].shape[1]
    grid = (t_n // block_t,)

    def rep(shape):
        return pl.BlockSpec(shape, lambda i: (0, 0))

    in_specs = [rep((1, s))] * 3 + [rep((s, c))] + \
        [pl.BlockSpec((block_t, 1), lambda i: (i, 0))] * 3 + \
        [pl.BlockSpec((block_t, cs), lambda i: (i, 0))] + \
        [rep(w0a.shape), rep(w0b.shape), rep((1, b0.shape[0]))]
    args = [pos_src[:, 0].reshape(1, s), pos_src[:, 1].reshape(1, s), pos_src[:, 2].reshape(1, s),
            x_src,
            pos_t[:, 0].reshape(t_n, 1), pos_t[:, 1].reshape(t_n, 1), pos_t[:, 2].reshape(t_n, 1),
            x_skip, w0a, w0b, b0.reshape(1, -1)]
    for (w_l, b_l) in rest:
        in_specs += [rep(w_l.shape), rep((1, b_l.shape[0]))]
        args += [w_l, b_l.reshape(1, -1)]

    return pl.pallas_call(
        functools.partial(_fp_knn3_kernel, n_layers=n_layers),
        grid=grid,
        in_specs=in_specs,
        out_specs=pl.BlockSpec((block_t, cout), lambda i: (i, 0)),
        out_shape=jax.ShapeDtypeStruct((t_n, cout), jnp.float32),
        interpret=interpret,
    )(*args)


def _sa_kernel(psx_ref, psy_ref, psz_ref, xh_ref, xl_ref, ptx_ref, pty_ref, ptz_ref,
               w0_ref, b0_ref, w1_ref, b1_ref, w2_ref, b2_ref, out_ref,
               d2_ref, acc_ref, any_ref, *, r2, k_nbr):
    dx = ptx_ref[:, :] - psx_ref[:, :]
    dy = pty_ref[:, :] - psy_ref[:, :]
    dz = ptz_ref[:, :] - psz_ref[:, :]
    d2_ref[:, :] = (dx * dx + dy * dy) + dz * dz
    acc_ref[:, :] = jnp.full(acc_ref.shape, -jnp.inf, jnp.float32)
    any_ref[:, :] = jnp.zeros(any_ref.shape, jnp.float32)

    xh = xh_ref[:, :]
    xl = xl_ref[:, :]
    w0 = w0_ref[:, :]
    off = (b0_ref[:, :]
           - (ptx_ref[:, :] * w0_ref[w0.shape[0] - 3:w0.shape[0] - 2, :]
              + pty_ref[:, :] * w0_ref[w0.shape[0] - 2:w0.shape[0] - 1, :]
              + ptz_ref[:, :] * w0_ref[w0.shape[0] - 1:w0.shape[0], :]))

    ex = 8

    def body(_, m_prev):
        d2c = d2_ref[:, :]
        ms = []
        mp = m_prev
        for _u in range(ex):
            mp = jnp.min(jnp.where(d2c > mp, d2c, jnp.inf), axis=1, keepdims=True)
            ms.append(mp)
        los = [m_prev] + ms[:-1]
        for m_lo, m_hi in zip(los, ms):
            selB = jnp.where((d2c == m_hi) & (d2c > m_lo), 1.0, 0.0).astype(jnp.bfloat16)
            feat = (jnp.dot(selB, xh, preferred_element_type=jnp.float32)
                    + jnp.dot(selB, xl, preferred_element_type=jnp.float32))
            h = jnp.maximum(_dot(feat, w0) + off, 0.0)
            h = jnp.maximum(_dot(h, w1_ref[:, :]) + b1_ref[:, :], 0.0)
            h = jnp.maximum(_dot(h, w2_ref[:, :]) + b2_ref[:, :], 0.0)
            valid = m_hi <= r2
            acc_ref[:, :] = jnp.where(valid, jnp.maximum(acc_ref[:, :], h), acc_ref[:, :])
            any_ref[:, :] = jnp.maximum(any_ref[:, :], valid.astype(jnp.float32))
        return ms[-1]

    jax.lax.fori_loop(0, k_nbr // ex, body,
                      jnp.full((out_ref.shape[0], 1), -jnp.inf, jnp.float32))
    out_ref[:, :] = jnp.where(any_ref[:, :] > 0.0, acc_ref[:, :], 0.0)


def _sa_pallas(x_feat, pos_src, pos_q, r, params, block_q, interpret=False):
    s = pos_src.shape[0]
    q = pos_q.shape[0]
    xpc = jnp.concatenate([x_feat, pos_src], axis=1)
    c_in = xpc.shape[1]
    (w0, b0), (w1, b1), (w2, b2) = params
    cout = w2.shape[1]
    grid = (q // block_q,)

    def rep(shape):
        return pl.BlockSpec(shape, lambda i: (0, 0))

    xh = xpc.astype(jnp.bfloat16)
    xl = (xpc - xh.astype(jnp.float32)).astype(jnp.bfloat16)
    in_specs = [rep((1, s))] * 3 + [rep((s, c_in))] * 2 + \
        [pl.BlockSpec((block_q, 1), lambda i: (i, 0))] * 3 + \
        [rep(w0.shape), rep((1, b0.shape[0])), rep(w1.shape), rep((1, b1.shape[0])),
         rep(w2.shape), rep((1, b2.shape[0]))]
    args = [pos_src[:, 0].reshape(1, s), pos_src[:, 1].reshape(1, s), pos_src[:, 2].reshape(1, s),
            xh, xl,
            pos_q[:, 0].reshape(q, 1), pos_q[:, 1].reshape(q, 1), pos_q[:, 2].reshape(q, 1),
            w0, b0.reshape(1, -1), w1, b1.reshape(1, -1), w2, b2.reshape(1, -1)]
    return pl.pallas_call(
        functools.partial(_sa_kernel, r2=r * r, k_nbr=K_NBR),
        grid=grid,
        in_specs=in_specs,
        out_specs=pl.BlockSpec((block_q, cout), lambda i: (i, 0)),
        out_shape=jax.ShapeDtypeStruct((q, cout), jnp.float32),
        scratch_shapes=[pltpu.VMEM((block_q, s), jnp.float32),
                        pltpu.VMEM((block_q, cout), jnp.float32),
                        pltpu.VMEM((block_q, 1), jnp.float32)],
        interpret=interpret,
    )(*args)


def _sa3_fp3_kernel(x2_ref, p2x_ref, p2y_ref, p2z_ref,
                    a0a_ref, a0b_ref, ab0_ref, a1_ref, ab1_ref, a2_ref, ab2_ref,
                    f0a_ref, f0b_ref, fb0_ref, f1_ref, fb1_ref,
                    out_ref):
    x2 = x2_ref[:, :]
    h = jnp.maximum(_dot(x2, a0a_ref[:, :])
                    + p2x_ref[:, :] * a0b_ref[0:1, :]
                    + p2y_ref[:, :] * a0b_ref[1:2, :]
                    + p2z_ref[:, :] * a0b_ref[2:3, :]
                    + ab0_ref[:, :], 0.0)
    h = jnp.maximum(_dot(h, a1_ref[:, :]) + ab1_ref[:, :], 0.0)
    h = jnp.maximum(_dot(h, a2_ref[:, :]) + ab2_ref[:, :], 0.0)
    x3 = jnp.max(h, axis=0, keepdims=True)
    g = jnp.maximum(_dot(x3, f0a_ref[:, :]) + _dot(x2, f0b_ref[:, :]) + fb0_ref[:, :], 0.0)
    g = jnp.maximum(_dot(g, f1_ref[:, :]) + fb1_ref[:, :], 0.0)
    out_ref[:, :] = g


def _sa3_fp3(x2, pos2, p_sa3, p_fp3, interpret=False):
    n2 = x2.shape[0]
    c2 = x2.shape[1]
    (a0, ab0), (a1, ab1), (a2, ab2) = p_sa3
    (f0, fb0), (f1, fb1) = p_fp3
    a0a, a0b = a0[:c2], a0[c2:]
    c3 = a2.shape[1]
    f0a, f0b = f0[:c3], f0[c3:]
    args = [x2,
            pos2[:, 0].reshape(n2, 1), pos2[:, 1].reshape(n2, 1), pos2[:, 2].reshape(n2, 1),
            a0a, a0b, ab0.reshape(1, -1), a1, ab1.reshape(1, -1), a2, ab2.reshape(1, -1),
            f0a, f0b, fb0.reshape(1, -1), f1, fb1.reshape(1, -1)]
    return pl.pallas_call(
        _sa3_fp3_kernel,
        out_shape=jax.ShapeDtypeStruct((n2, f1.shape[1]), jnp.float32),
        interpret=interpret,
    )(*args)


def _radius(pos_src, pos_q, r, K):
    d2 = jnp.sum((pos_q[:, None, :] - pos_src[None, :, :]) ** 2, axis=-1)
    neg, idx = jax.lax.top_k(-d2, K)
    return idx, (-neg) <= r * r


def _knn_idx(pos_src, pos_t, k):
    d2 = jnp.sum((pos_t[:, None, :] - pos_src[None, :, :]) ** 2, axis=-1)
    _, idx = jax.lax.top_k(-d2, min(k, pos_src.shape[0]))
    return idx


def _sa(x, pos_src, pos_q, nidx, valid, params):
    h = jnp.concatenate([x[nidx], pos_src[nidx] - pos_q[:, None, :]], axis=-1)
    h = _mlp(h, params)
    h = jnp.where(valid[..., None], h, -jnp.inf)
    out = jnp.max(h, axis=1)
    return jnp.where(jnp.any(valid, axis=1)[:, None], out, 0.0)


def _fp(x_src, pos_src, pos_t, kidx, x_skip, params):
    d2 = jnp.sum((pos_t[:, None, :] - pos_src[kidx]) ** 2, axis=-1)
    w = 1.0 / jnp.clip(d2, 1e-16, None)
    y = jnp.sum(x_src[kidx] * w[..., None], axis=1) / jnp.sum(w, axis=1, keepdims=True)
    y = jnp.concatenate([y, x_skip], axis=1)
    return _mlp(y, params)


def kernel(x, pos, batch,
           sa1_w0, sa1_b0, sa1_w1, sa1_b1, sa1_w2, sa1_b2,
           sa2_w0, sa2_b0, sa2_w1, sa2_b1, sa2_w2, sa2_b2,
           sa3_w0, sa3_b0, sa3_w1, sa3_b1, sa3_w2, sa3_b2,
           fp3_w0, fp3_b0, fp3_w1, fp3_b1,
           fp2_w0, fp2_b0, fp2_w1, fp2_b1,
           fp1_w0, fp1_b0, fp1_w1, fp1_b1, fp1_w2, fp1_b2):
    inp = dict(locals())
    def params(name, n):
        return tuple((inp[name + '_w' + str(i)], inp[name + '_b' + str(i)]) for i in range(n))
    pos = pos + batch.astype(pos.dtype)[:, None]
    p_sa1, p_sa2, p_sa3 = params('sa1', 3), params('sa2', 3), params('sa3', 3)
    p_fp3, p_fp2, p_fp1 = params('fp3', 2), params('fp2', 2), params('fp1', 3)
    pos1 = _fps_pos(pos, N_PTS // 2)
    x1 = _sa_pallas(x, pos, pos1, 0.2, p_sa1, block_q=256)
    pos2 = _fps_pos(pos1, pos1.shape[0] // 4)
    x2 = _sa_pallas(x1, pos1, pos2, 0.4, p_sa2, block_q=256)
    f3 = _sa3_fp3(x2, pos2, p_sa3, p_fp3)
    f2 = _fp_knn3(pos2, f3, pos1, x1, p_fp2, block_t=512)
    f1 = _fp_knn3(pos1, f2, pos, x, p_fp1, block_t=512)
    return f1


# speedup vs baseline: 1.1274x; 1.1274x over previous
import functools

import jax
import jax.numpy as jnp
from jax.experimental import pallas as pl
from jax.experimental.pallas import tpu as pltpu

IN_DIM = 6
N_PTS = 4096
K_NBR = 64

_BIG_I = 1 << 30


def _fps_kernel(px_ref, py_ref, pz_ref, ox_ref, oy_ref, oz_ref, *, n_samples):
    px = px_ref[:, :]
    py = py_ref[:, :]
    pz = pz_ref[:, :]
    shape = px.shape
    iota = jax.lax.broadcasted_iota(jnp.int32, shape, 0) * 128 + jax.lax.broadcasted_iota(jnp.int32, shape, 1)

    x0 = px[0, 0]
    y0 = py[0, 0]
    z0 = pz[0, 0]
    ox_ref[0:1, :] = x0.reshape(1, 1)
    oy_ref[0:1, :] = y0.reshape(1, 1)
    oz_ref[0:1, :] = z0.reshape(1, 1)
    dx = px - x0
    dy = py - y0
    dz = pz - z0
    d = (dx * dx + dy * dy) + dz * dz

    def body(i, d):
        m = jnp.max(d)
        nxt = jnp.min(jnp.where(d == m, iota, _BIG_I))
        msk = iota == nxt
        xn = jnp.sum(jnp.where(msk, px, 0.0))
        yn = jnp.sum(jnp.where(msk, py, 0.0))
        zn = jnp.sum(jnp.where(msk, pz, 0.0))
        ox_ref[pl.ds(i, 1), :] = xn.reshape(1, 1)
        oy_ref[pl.ds(i, 1), :] = yn.reshape(1, 1)
        oz_ref[pl.ds(i, 1), :] = zn.reshape(1, 1)
        ex = px - xn
        ey = py - yn
        ez = pz - zn
        nd = (ex * ex + ey * ey) + ez * ez
        return jnp.minimum(d, nd)

    jax.lax.fori_loop(1, n_samples, body, d)


def _fps_pos(pos, n_samples, interpret=False):
    n = pos.shape[0]
    r = n // 128
    px = pos[:, 0].reshape(r, 128)
    py = pos[:, 1].reshape(r, 128)
    pz = pos[:, 2].reshape(r, 128)
    out = pl.pallas_call(
        functools.partial(_fps_kernel, n_samples=n_samples),
        out_shape=[jax.ShapeDtypeStruct((n_samples, 1), jnp.float32)] * 3,
        interpret=interpret,
    )(px, py, pz)
    return jnp.concatenate(out, axis=1)


def _mlp(h, params):
    for W, b in params:
        h = jax.nn.relu(h @ W + b)
    return h


def _dot(a, b):
    return jnp.dot(a, b, preferred_element_type=jnp.float32,
                   precision=jax.lax.Precision.HIGHEST)


def _fp_knn3_kernel(*refs, n_layers):
    (psx_ref, psy_ref, psz_ref, xsrc_ref, ptx_ref, pty_ref, ptz_ref,
     xskip_ref, w0a_ref, w0b_ref, b0_ref) = refs[:11]
    layer_refs = refs[11:11 + 2 * (n_layers - 1)]
    out_ref = refs[11 + 2 * (n_layers - 1)]

    dx = ptx_ref[:, :] - psx_ref[:, :]
    dy = pty_ref[:, :] - psy_ref[:, :]
    dz = ptz_ref[:, :] - psz_ref[:, :]
    d2 = (dx * dx + dy * dy) + dz * dz
    m1 = jnp.min(d2, axis=1, keepdims=True)
    t = jnp.where(d2 == m1, jnp.inf, d2)
    m2 = jnp.min(t, axis=1, keepdims=True)
    t = jnp.where(t == m2, jnp.inf, t)
    m3 = jnp.min(t, axis=1, keepdims=True)
    w = jnp.where(d2 <= m3, 1.0 / jnp.maximum(d2, 1e-16), 0.0)
    sw = jnp.sum(w, axis=1, keepdims=True)
    y = _dot(w, xsrc_ref[:, :]) / sw
    h = jnp.maximum(_dot(y, w0a_ref[:, :]) + _dot(xskip_ref[:, :], w0b_ref[:, :]) + b0_ref[:, :], 0.0)
    for li in range(n_layers - 1):
        w_ref, b_ref = layer_refs[2 * li], layer_refs[2 * li + 1]
        h = jnp.maximum(_dot(h, w_ref[:, :]) + b_ref[:, :], 0.0)
    out_ref[:, :] = h


def _fp_knn3(pos_src, x_src, pos_t, x_skip, params, block_t, interpret=False):
    s = pos_src.shape[0]
    t_n = pos_t.shape[0]
    c = x_src.shape[1]
    cs = x_skip.shape[1]
    w0, b0 = params[0]
    w0a, w0b = w0[:c], w0[c:]
    rest = params[1:]
    n_layers = len(params)
    cout = params[-1][0].shape[1]
    grid = (t_n // block_t,)

    def rep(shape):
        return pl.BlockSpec(shape, lambda i: (0, 0))

    in_specs = [rep((1, s))] * 3 + [rep((s, c))] + \
        [pl.BlockSpec((block_t, 1), lambda i: (i, 0))] * 3 + \
        [pl.BlockSpec((block_t, cs), lambda i: (i, 0))] + \
        [rep(w0a.shape), rep(w0b.shape), rep((1, b0.shape[0]))]
    args = [pos_src[:, 0].reshape(1, s), pos_src[:, 1].reshape(1, s), pos_src[:, 2].reshape(1, s),
            x_src,
            pos_t[:, 0].reshape(t_n, 1), pos_t[:, 1].reshape(t_n, 1), pos_t[:, 2].reshape(t_n, 1),
            x_skip, w0a, w0b, b0.reshape(1, -1)]
    for (w_l, b_l) in rest:
        in_specs += [rep(w_l.shape), rep((1, b_l.shape[0]))]
        args += [w_l, b_l.reshape(1, -1)]

    return pl.pallas_call(
        functools.partial(_fp_knn3_kernel, n_layers=n_layers),
        grid=grid,
        in_specs=in_specs,
        out_specs=pl.BlockSpec((block_t, cout), lambda i: (i, 0)),
        out_shape=jax.ShapeDtypeStruct((t_n, cout), jnp.float32),
        interpret=interpret,
    )(*args)


def _sa_kernel(psx_ref, psy_ref, psz_ref, xh_ref, xl_ref, ptx_ref, pty_ref, ptz_ref,
               w0_ref, b0_ref, w1_ref, b1_ref, w2_ref, b2_ref, out_ref,
               d2_ref, acc_ref, any_ref, *, r2, k_nbr):
    dx = ptx_ref[:, :] - psx_ref[:, :]
    dy = pty_ref[:, :] - psy_ref[:, :]
    dz = ptz_ref[:, :] - psz_ref[:, :]
    d2_ref[:, :] = (dx * dx + dy * dy) + dz * dz
    acc_ref[:, :] = jnp.full(acc_ref.shape, -jnp.inf, jnp.float32)
    any_ref[:, :] = jnp.zeros(any_ref.shape, jnp.float32)

    xh = xh_ref[:, :]
    xl = xl_ref[:, :]
    w0 = w0_ref[:, :]
    off = (b0_ref[:, :]
           - (ptx_ref[:, :] * w0_ref[w0.shape[0] - 3:w0.shape[0] - 2, :]
              + pty_ref[:, :] * w0_ref[w0.shape[0] - 2:w0.shape[0] - 1, :]
              + ptz_ref[:, :] * w0_ref[w0.shape[0] - 1:w0.shape[0], :]))

    ex = 4

    def body(_, m_prev):
        d2c = d2_ref[:, :]
        ms = []
        mp = m_prev
        for _u in range(ex):
            mp = jnp.min(jnp.where(d2c > mp, d2c, jnp.inf), axis=1, keepdims=True)
            ms.append(mp)
        los = [m_prev] + ms[:-1]
        for m_lo, m_hi in zip(los, ms):
            selB = jnp.where((d2c == m_hi) & (d2c > m_lo), 1.0, 0.0).astype(jnp.bfloat16)
            feat = (jnp.dot(selB, xh, preferred_element_type=jnp.float32)
                    + jnp.dot(selB, xl, preferred_element_type=jnp.float32))
            h = jnp.maximum(jnp.dot(feat, w0, preferred_element_type=jnp.float32) + off, 0.0)
            h = jnp.maximum(jnp.dot(h, w1_ref[:, :], preferred_element_type=jnp.float32) + b1_ref[:, :], 0.0)
            h = jnp.maximum(jnp.dot(h, w2_ref[:, :], preferred_element_type=jnp.float32) + b2_ref[:, :], 0.0)
            valid = m_hi <= r2
            acc_ref[:, :] = jnp.where(valid, jnp.maximum(acc_ref[:, :], h), acc_ref[:, :])
            any_ref[:, :] = jnp.maximum(any_ref[:, :], valid.astype(jnp.float32))
        return ms[-1]

    jax.lax.fori_loop(0, k_nbr // ex, body,
                      jnp.full((out_ref.shape[0], 1), -jnp.inf, jnp.float32))
    out_ref[:, :] = jnp.where(any_ref[:, :] > 0.0, acc_ref[:, :], 0.0)


def _sa_pallas(x_feat, pos_src, pos_q, r, params, block_q, interpret=False):
    s = pos_src.shape[0]
    q = pos_q.shape[0]
    xpc = jnp.concatenate([x_feat, pos_src], axis=1)
    c_in = xpc.shape[1]
    (w0, b0), (w1, b1), (w2, b2) = params
    cout = w2.shape[1]
    grid = (q // block_q,)

    def rep(shape):
        return pl.BlockSpec(shape, lambda i: (0, 0))

    xh = xpc.astype(jnp.bfloat16)
    xl = (xpc - xh.astype(jnp.float32)).astype(jnp.bfloat16)
    in_specs = [rep((1, s))] * 3 + [rep((s, c_in))] * 2 + \
        [pl.BlockSpec((block_q, 1), lambda i: (i, 0))] * 3 + \
        [rep(w0.shape), rep((1, b0.shape[0])), rep(w1.shape), rep((1, b1.shape[0])),
         rep(w2.shape), rep((1, b2.shape[0]))]
    args = [pos_src[:, 0].reshape(1, s), pos_src[:, 1].reshape(1, s), pos_src[:, 2].reshape(1, s),
            xh, xl,
            pos_q[:, 0].reshape(q, 1), pos_q[:, 1].reshape(q, 1), pos_q[:, 2].reshape(q, 1),
            w0, b0.reshape(1, -1), w1, b1.reshape(1, -1), w2, b2.reshape(1, -1)]
    return pl.pallas_call(
        functools.partial(_sa_kernel, r2=r * r, k_nbr=K_NBR),
        grid=grid,
        in_specs=in_specs,
        out_specs=pl.BlockSpec((block_q, cout), lambda i: (i, 0)),
        out_shape=jax.ShapeDtypeStruct((q, cout), jnp.float32),
        scratch_shapes=[pltpu.VMEM((block_q, s), jnp.float32),
                        pltpu.VMEM((block_q, cout), jnp.float32),
                        pltpu.VMEM((block_q, 1), jnp.float32)],
        interpret=interpret,
    )(*args)


def _sa3_fp3_kernel(x2_ref, p2x_ref, p2y_ref, p2z_ref,
                    a0a_ref, a0b_ref, ab0_ref, a1_ref, ab1_ref, a2_ref, ab2_ref,
                    f0a_ref, f0b_ref, fb0_ref, f1_ref, fb1_ref,
                    out_ref):
    x2 = x2_ref[:, :]
    h = jnp.maximum(_dot(x2, a0a_ref[:, :])
                    + p2x_ref[:, :] * a0b_ref[0:1, :]
                    + p2y_ref[:, :] * a0b_ref[1:2, :]
                    + p2z_ref[:, :] * a0b_ref[2:3, :]
                    + ab0_ref[:, :], 0.0)
    h = jnp.maximum(_dot(h, a1_ref[:, :]) + ab1_ref[:, :], 0.0)
    h = jnp.maximum(_dot(h, a2_ref[:, :]) + ab2_ref[:, :], 0.0)
    x3 = jnp.max(h, axis=0, keepdims=True)
    g = jnp.maximum(_dot(x3, f0a_ref[:, :]) + _dot(x2, f0b_ref[:, :]) + fb0_ref[:, :], 0.0)
    g = jnp.maximum(_dot(g, f1_ref[:, :]) + fb1_ref[:, :], 0.0)
    out_ref[:, :] = g


def _sa3_fp3(x2, pos2, p_sa3, p_fp3, interpret=False):
    n2 = x2.shape[0]
    c2 = x2.shape[1]
    (a0, ab0), (a1, ab1), (a2, ab2) = p_sa3
    (f0, fb0), (f1, fb1) = p_fp3
    a0a, a0b = a0[:c2], a0[c2:]
    c3 = a2.shape[1]
    f0a, f0b = f0[:c3], f0[c3:]
    args = [x2,
            pos2[:, 0].reshape(n2, 1), pos2[:, 1].reshape(n2, 1), pos2[:, 2].reshape(n2, 1),
            a0a, a0b, ab0.reshape(1, -1), a1, ab1.reshape(1, -1), a2, ab2.reshape(1, -1),
            f0a, f0b, fb0.reshape(1, -1), f1, fb1.reshape(1, -1)]
    return pl.pallas_call(
        _sa3_fp3_kernel,
        out_shape=jax.ShapeDtypeStruct((n2, f1.shape[1]), jnp.float32),
        interpret=interpret,
    )(*args)


def _radius(pos_src, pos_q, r, K):
    d2 = jnp.sum((pos_q[:, None, :] - pos_src[None, :, :]) ** 2, axis=-1)
    neg, idx = jax.lax.top_k(-d2, K)
    return idx, (-neg) <= r * r


def _knn_idx(pos_src, pos_t, k):
    d2 = jnp.sum((pos_t[:, None, :] - pos_src[None, :, :]) ** 2, axis=-1)
    _, idx = jax.lax.top_k(-d2, min(k, pos_src.shape[0]))
    return idx


def _sa(x, pos_src, pos_q, nidx, valid, params):
    h = jnp.concatenate([x[nidx], pos_src[nidx] - pos_q[:, None, :]], axis=-1)
    h = _mlp(h, params)
    h = jnp.where(valid[..., None], h, -jnp.inf)
    out = jnp.max(h, axis=1)
    return jnp.where(jnp.any(valid, axis=1)[:, None], out, 0.0)


def _fp(x_src, pos_src, pos_t, kidx, x_skip, params):
    d2 = jnp.sum((pos_t[:, None, :] - pos_src[kidx]) ** 2, axis=-1)
    w = 1.0 / jnp.clip(d2, 1e-16, None)
    y = jnp.sum(x_src[kidx] * w[..., None], axis=1) / jnp.sum(w, axis=1, keepdims=True)
    y = jnp.concatenate([y, x_skip], axis=1)
    return _mlp(y, params)


def kernel(x, pos, batch,
           sa1_w0, sa1_b0, sa1_w1, sa1_b1, sa1_w2, sa1_b2,
           sa2_w0, sa2_b0, sa2_w1, sa2_b1, sa2_w2, sa2_b2,
           sa3_w0, sa3_b0, sa3_w1, sa3_b1, sa3_w2, sa3_b2,
           fp3_w0, fp3_b0, fp3_w1, fp3_b1,
           fp2_w0, fp2_b0, fp2_w1, fp2_b1,
           fp1_w0, fp1_b0, fp1_w1, fp1_b1, fp1_w2, fp1_b2):
    inp = dict(locals())
    def params(name, n):
        return tuple((inp[name + '_w' + str(i)], inp[name + '_b' + str(i)]) for i in range(n))
    pos = pos + batch.astype(pos.dtype)[:, None]
    p_sa1, p_sa2, p_sa3 = params('sa1', 3), params('sa2', 3), params('sa3', 3)
    p_fp3, p_fp2, p_fp1 = params('fp3', 2), params('fp2', 2), params('fp1', 3)
    pos1 = _fps_pos(pos, N_PTS // 2)
    x1 = _sa_pallas(x, pos, pos1, 0.2, p_sa1, block_q=256)
    pos2 = _fps_pos(pos1, pos1.shape[0] // 4)
    x2 = _sa_pallas(x1, pos1, pos2, 0.4, p_sa2, block_q=256)
    f3 = _sa3_fp3(x2, pos2, p_sa3, p_fp3)
    f2 = _fp_knn3(pos2, f3, pos1, x1, p_fp2, block_t=512)
    f1 = _fp_knn3(pos1, f2, pos, x, p_fp1, block_t=512)
    return f1


# speedup vs baseline: 1.2861x; 1.1408x over previous
import functools

import jax
import jax.numpy as jnp
from jax.experimental import pallas as pl
from jax.experimental.pallas import tpu as pltpu

IN_DIM = 6
N_PTS = 4096
K_NBR = 64

_BIG_I = 1 << 30


def _fps_kernel(px_ref, py_ref, pz_ref, ox_ref, oy_ref, oz_ref, *, n_samples):
    px = px_ref[:, :]
    py = py_ref[:, :]
    pz = pz_ref[:, :]
    shape = px.shape
    iota = jax.lax.broadcasted_iota(jnp.int32, shape, 0) * 128 + jax.lax.broadcasted_iota(jnp.int32, shape, 1)

    x0 = px[0, 0]
    y0 = py[0, 0]
    z0 = pz[0, 0]
    ox_ref[0:1, :] = x0.reshape(1, 1)
    oy_ref[0:1, :] = y0.reshape(1, 1)
    oz_ref[0:1, :] = z0.reshape(1, 1)
    dx = px - x0
    dy = py - y0
    dz = pz - z0
    d = (dx * dx + dy * dy) + dz * dz

    def body(i, d):
        m = jnp.max(d)
        nxt = jnp.min(jnp.where(d == m, iota, _BIG_I))
        msk = iota == nxt
        xn = jnp.sum(jnp.where(msk, px, 0.0))
        yn = jnp.sum(jnp.where(msk, py, 0.0))
        zn = jnp.sum(jnp.where(msk, pz, 0.0))
        ox_ref[pl.ds(i, 1), :] = xn.reshape(1, 1)
        oy_ref[pl.ds(i, 1), :] = yn.reshape(1, 1)
        oz_ref[pl.ds(i, 1), :] = zn.reshape(1, 1)
        ex = px - xn
        ey = py - yn
        ez = pz - zn
        nd = (ex * ex + ey * ey) + ez * ez
        return jnp.minimum(d, nd)

    jax.lax.fori_loop(1, n_samples, body, d)


def _fps_pos(pos, n_samples, interpret=False):
    n = pos.shape[0]
    r = n // 128
    px = pos[:, 0].reshape(r, 128)
    py = pos[:, 1].reshape(r, 128)
    pz = pos[:, 2].reshape(r, 128)
    out = pl.pallas_call(
        functools.partial(_fps_kernel, n_samples=n_samples),
        out_shape=[jax.ShapeDtypeStruct((n_samples, 1), jnp.float32)] * 3,
        interpret=interpret,
    )(px, py, pz)
    return jnp.concatenate(out, axis=1)


def _mlp(h, params):
    for W, b in params:
        h = jax.nn.relu(h @ W + b)
    return h


def _dot(a, b):
    return jnp.dot(a, b, preferred_element_type=jnp.float32,
                   precision=jax.lax.Precision.HIGHEST)


def _fp_knn3_kernel(*refs, n_layers):
    (psx_ref, psy_ref, psz_ref, xsrc_ref, ptx_ref, pty_ref, ptz_ref,
     xskip_ref, w0a_ref, w0b_ref, b0_ref) = refs[:11]
    layer_refs = refs[11:11 + 2 * (n_layers - 1)]
    out_ref = refs[11 + 2 * (n_layers - 1)]

    dx = ptx_ref[:, :] - psx_ref[:, :]
    dy = pty_ref[:, :] - psy_ref[:, :]
    dz = ptz_ref[:, :] - psz_ref[:, :]
    d2 = (dx * dx + dy * dy) + dz * dz
    m1 = jnp.min(d2, axis=1, keepdims=True)
    t = jnp.where(d2 == m1, jnp.inf, d2)
    m2 = jnp.min(t, axis=1, keepdims=True)
    t = jnp.where(t == m2, jnp.inf, t)
    m3 = jnp.min(t, axis=1, keepdims=True)
    w = jnp.where(d2 <= m3, 1.0 / jnp.maximum(d2, 1e-16), 0.0)
    sw = jnp.sum(w, axis=1, keepdims=True)
    y = _dot(w, xsrc_ref[:, :]) / sw
    h = jnp.maximum(_dot(y, w0a_ref[:, :]) + _dot(xskip_ref[:, :], w0b_ref[:, :]) + b0_ref[:, :], 0.0)
    for li in range(n_layers - 1):
        w_ref, b_ref = layer_refs[2 * li], layer_refs[2 * li + 1]
        h = jnp.maximum(_dot(h, w_ref[:, :]) + b_ref[:, :], 0.0)
    out_ref[:, :] = h


def _fp_knn3(pos_src, x_src, pos_t, x_skip, params, block_t, interpret=False):
    s = pos_src.shape[0]
    t_n = pos_t.shape[0]
    c = x_src.shape[1]
    cs = x_skip.shape[1]
    w0, b0 = params[0]
    w0a, w0b = w0[:c], w0[c:]
    rest = params[1:]
    n_layers = len(params)
    cout = params[-1][0].shape[1]
    grid = (t_n // block_t,)

    def rep(shape):
        return pl.BlockSpec(shape, lambda i: (0, 0))

    in_specs = [rep((1, s))] * 3 + [rep((s, c))] + \
        [pl.BlockSpec((block_t, 1), lambda i: (i, 0))] * 3 + \
        [pl.BlockSpec((block_t, cs), lambda i: (i, 0))] + \
        [rep(w0a.shape), rep(w0b.shape), rep((1, b0.shape[0]))]
    args = [pos_src[:, 0].reshape(1, s), pos_src[:, 1].reshape(1, s), pos_src[:, 2].reshape(1, s),
            x_src,
            pos_t[:, 0].reshape(t_n, 1), pos_t[:, 1].reshape(t_n, 1), pos_t[:, 2].reshape(t_n, 1),
            x_skip, w0a, w0b, b0.reshape(1, -1)]
    for (w_l, b_l) in rest:
        in_specs += [rep(w_l.shape), rep((1, b_l.shape[0]))]
        args += [w_l, b_l.reshape(1, -1)]

    return pl.pallas_call(
        functools.partial(_fp_knn3_kernel, n_layers=n_layers),
        grid=grid,
        in_specs=in_specs,
        out_specs=pl.BlockSpec((block_t, cout), lambda i: (i, 0)),
        out_shape=jax.ShapeDtypeStruct((t_n, cout), jnp.float32),
        interpret=interpret,
    )(*args)


def _sa_kernel(psx_ref, psy_ref, psz_ref, xh_ref, xl_ref, ptx_ref, pty_ref, ptz_ref,
               w0_ref, b0_ref, w1_ref, b1_ref, w2_ref, b2_ref, out_ref,
               d2_ref, acc_ref, any_ref, *, r2, k_nbr):
    dx = ptx_ref[:, :] - psx_ref[:, :]
    dy = pty_ref[:, :] - psy_ref[:, :]
    dz = ptz_ref[:, :] - psz_ref[:, :]
    d2_ref[:, :] = (dx * dx + dy * dy) + dz * dz
    acc_ref[:, :] = jnp.full(acc_ref.shape, -jnp.inf, jnp.float32)
    any_ref[:, :] = jnp.zeros(any_ref.shape, jnp.float32)

    xh = xh_ref[:, :]
    xl = xl_ref[:, :]
    w0 = w0_ref[:, :]
    off = (b0_ref[:, :]
           - (ptx_ref[:, :] * w0_ref[w0.shape[0] - 3:w0.shape[0] - 2, :]
              + pty_ref[:, :] * w0_ref[w0.shape[0] - 2:w0.shape[0] - 1, :]
              + ptz_ref[:, :] * w0_ref[w0.shape[0] - 1:w0.shape[0], :]))

    ex = 4
    qb = out_ref.shape[0]
    off4 = jnp.concatenate([off] * ex, axis=0)

    def body(_, m_prev):
        d2c = d2_ref[:, :]
        ms = []
        mp = m_prev
        for _u in range(ex):
            mp = jnp.min(jnp.where(d2c > mp, d2c, jnp.inf), axis=1, keepdims=True)
            ms.append(mp)
        los = [m_prev] + ms[:-1]
        sel4 = jnp.concatenate(
            [jnp.where((d2c == m_hi) & (d2c > m_lo), 1.0, 0.0).astype(jnp.bfloat16)
             for m_lo, m_hi in zip(los, ms)], axis=0)
        feat = (jnp.dot(sel4, xh, preferred_element_type=jnp.float32)
                + jnp.dot(sel4, xl, preferred_element_type=jnp.float32))
        h = jnp.maximum(jnp.dot(feat, w0, preferred_element_type=jnp.float32) + off4, 0.0)
        h = jnp.maximum(jnp.dot(h, w1_ref[:, :], preferred_element_type=jnp.float32) + b1_ref[:, :], 0.0)
        h = jnp.maximum(jnp.dot(h, w2_ref[:, :], preferred_element_type=jnp.float32) + b2_ref[:, :], 0.0)
        valid4 = jnp.concatenate(ms, axis=0) <= r2
        hm = jnp.where(valid4, h, -jnp.inf)
        f = jnp.maximum(jnp.maximum(hm[0:qb], hm[qb:2 * qb]),
                        jnp.maximum(hm[2 * qb:3 * qb], hm[3 * qb:4 * qb]))
        acc_ref[:, :] = jnp.maximum(acc_ref[:, :], f)
        any_ref[:, :] = jnp.maximum(any_ref[:, :], (ms[0] <= r2).astype(jnp.float32))
        return ms[-1]

    jax.lax.fori_loop(0, k_nbr // ex, body,
                      jnp.full((out_ref.shape[0], 1), -jnp.inf, jnp.float32))
    out_ref[:, :] = jnp.where(any_ref[:, :] > 0.0, acc_ref[:, :], 0.0)


def _sa_pallas(x_feat, pos_src, pos_q, r, params, block_q, interpret=False):
    s = pos_src.shape[0]
    q = pos_q.shape[0]
    xpc = jnp.concatenate([x_feat, pos_src], axis=1)
    c_in = xpc.shape[1]
    (w0, b0), (w1, b1), (w2, b2) = params
    cout = w2.shape[1]
    grid = (q // block_q,)

    def rep(shape):
        return pl.BlockSpec(shape, lambda i: (0, 0))

    xh = xpc.astype(jnp.bfloat16)
    xl = (xpc - xh.astype(jnp.float32)).astype(jnp.bfloat16)
    in_specs = [rep((1, s))] * 3 + [rep((s, c_in))] * 2 + \
        [pl.BlockSpec((block_q, 1), lambda i: (i, 0))] * 3 + \
        [rep(w0.shape), rep((1, b0.shape[0])), rep(w1.shape), rep((1, b1.shape[0])),
         rep(w2.shape), rep((1, b2.shape[0]))]
    args = [pos_src[:, 0].reshape(1, s), pos_src[:, 1].reshape(1, s), pos_src[:, 2].reshape(1, s),
            xh, xl,
            pos_q[:, 0].reshape(q, 1), pos_q[:, 1].reshape(q, 1), pos_q[:, 2].reshape(q, 1),
            w0, b0.reshape(1, -1), w1, b1.reshape(1, -1), w2, b2.reshape(1, -1)]
    return pl.pallas_call(
        functools.partial(_sa_kernel, r2=r * r, k_nbr=K_NBR),
        grid=grid,
        in_specs=in_specs,
        out_specs=pl.BlockSpec((block_q, cout), lambda i: (i, 0)),
        out_shape=jax.ShapeDtypeStruct((q, cout), jnp.float32),
        scratch_shapes=[pltpu.VMEM((block_q, s), jnp.float32),
                        pltpu.VMEM((block_q, cout), jnp.float32),
                        pltpu.VMEM((block_q, 1), jnp.float32)],
        interpret=interpret,
    )(*args)


def _sa3_fp3_kernel(x2_ref, p2x_ref, p2y_ref, p2z_ref,
                    a0a_ref, a0b_ref, ab0_ref, a1_ref, ab1_ref, a2_ref, ab2_ref,
                    f0a_ref, f0b_ref, fb0_ref, f1_ref, fb1_ref,
                    out_ref):
    x2 = x2_ref[:, :]
    h = jnp.maximum(_dot(x2, a0a_ref[:, :])
                    + p2x_ref[:, :] * a0b_ref[0:1, :]
                    + p2y_ref[:, :] * a0b_ref[1:2, :]
                    + p2z_ref[:, :] * a0b_ref[2:3, :]
                    + ab0_ref[:, :], 0.0)
    h = jnp.maximum(_dot(h, a1_ref[:, :]) + ab1_ref[:, :], 0.0)
    h = jnp.maximum(_dot(h, a2_ref[:, :]) + ab2_ref[:, :], 0.0)
    x3 = jnp.max(h, axis=0, keepdims=True)
    g = jnp.maximum(_dot(x3, f0a_ref[:, :]) + _dot(x2, f0b_ref[:, :]) + fb0_ref[:, :], 0.0)
    g = jnp.maximum(_dot(g, f1_ref[:, :]) + fb1_ref[:, :], 0.0)
    out_ref[:, :] = g


def _sa3_fp3(x2, pos2, p_sa3, p_fp3, interpret=False):
    n2 = x2.shape[0]
    c2 = x2.shape[1]
    (a0, ab0), (a1, ab1), (a2, ab2) = p_sa3
    (f0, fb0), (f1, fb1) = p_fp3
    a0a, a0b = a0[:c2], a0[c2:]
    c3 = a2.shape[1]
    f0a, f0b = f0[:c3], f0[c3:]
    args = [x2,
            pos2[:, 0].reshape(n2, 1), pos2[:, 1].reshape(n2, 1), pos2[:, 2].reshape(n2, 1),
            a0a, a0b, ab0.reshape(1, -1), a1, ab1.reshape(1, -1), a2, ab2.reshape(1, -1),
            f0a, f0b, fb0.reshape(1, -1), f1, fb1.reshape(1, -1)]
    return pl.pallas_call(
        _sa3_fp3_kernel,
        out_shape=jax.ShapeDtypeStruct((n2, f1.shape[1]), jnp.float32),
        interpret=interpret,
    )(*args)


def _radius(pos_src, pos_q, r, K):
    d2 = jnp.sum((pos_q[:, None, :] - pos_src[None, :, :]) ** 2, axis=-1)
    neg, idx = jax.lax.top_k(-d2, K)
    return idx, (-neg) <= r * r


def _knn_idx(pos_src, pos_t, k):
    d2 = jnp.sum((pos_t[:, None, :] - pos_src[None, :, :]) ** 2, axis=-1)
    _, idx = jax.lax.top_k(-d2, min(k, pos_src.shape[0]))
    return idx


def _sa(x, pos_src, pos_q, nidx, valid, params):
    h = jnp.concatenate([x[nidx], pos_src[nidx] - pos_q[:, None, :]], axis=-1)
    h = _mlp(h, params)
    h = jnp.where(valid[..., None], h, -jnp.inf)
    out = jnp.max(h, axis=1)
    return jnp.where(jnp.any(valid, axis=1)[:, None], out, 0.0)


def _fp(x_src, pos_src, pos_t, kidx, x_skip, params):
    d2 = jnp.sum((pos_t[:, None, :] - pos_src[kidx]) ** 2, axis=-1)
    w = 1.0 / jnp.clip(d2, 1e-16, None)
    y = jnp.sum(x_src[kidx] * w[..., None], axis=1) / jnp.sum(w, axis=1, keepdims=True)
    y = jnp.concatenate([y, x_skip], axis=1)
    return _mlp(y, params)


def kernel(x, pos, batch,
           sa1_w0, sa1_b0, sa1_w1, sa1_b1, sa1_w2, sa1_b2,
           sa2_w0, sa2_b0, sa2_w1, sa2_b1, sa2_w2, sa2_b2,
           sa3_w0, sa3_b0, sa3_w1, sa3_b1, sa3_w2, sa3_b2,
           fp3_w0, fp3_b0, fp3_w1, fp3_b1,
           fp2_w0, fp2_b0, fp2_w1, fp2_b1,
           fp1_w0, fp1_b0, fp1_w1, fp1_b1, fp1_w2, fp1_b2):
    inp = dict(locals())
    def params(name, n):
        return tuple((inp[name + '_w' + str(i)], inp[name + '_b' + str(i)]) for i in range(n))
    pos = pos + batch.astype(pos.dtype)[:, None]
    p_sa1, p_sa2, p_sa3 = params('sa1', 3), params('sa2', 3), params('sa3', 3)
    p_fp3, p_fp2, p_fp1 = params('fp3', 2), params('fp2', 2), params('fp1', 3)
    pos1 = _fps_pos(pos, N_PTS // 2)
    x1 = _sa_pallas(x, pos, pos1, 0.2, p_sa1, block_q=256)
    pos2 = _fps_pos(pos1, pos1.shape[0] // 4)
    x2 = _sa_pallas(x1, pos1, pos2, 0.4, p_sa2, block_q=256)
    f3 = _sa3_fp3(x2, pos2, p_sa3, p_fp3)
    f2 = _fp_knn3(pos2, f3, pos1, x1, p_fp2, block_t=512)
    f1 = _fp_knn3(pos1, f2, pos, x, p_fp1, block_t=512)
    return f1
